# baseline (speedup 1.0000x reference)
import functools

import jax
import jax.numpy as jnp
from jax import lax
from jax.experimental import pallas as pl
from jax.experimental.pallas import tpu as pltpu
from jax.experimental.pallas import tpu_sc as plsc

_N = 10000
_D = 128
_E = 320000
_DE = 16
_CH = 128
_NCHUNK = 2560
_EPAD = _NCHUNK * _CH
_NC, _NS = 2, 16
_NTILES = _NC * _NS
_CPT = _NCHUNK // _NTILES
_NPAD = 10240
_STRIPE = _NPAD // _NS


def _mesh():
    return plsc.VectorSubcoreMesh(
        core_axis_name="c", subcore_axis_name="s",
        num_cores=_NC, num_subcores=_NS)


def _seg3_body(xn_hbm, xo_hbm, si_nn, di_nn, si_on, di_on, si_oo, di_oo,
               out_hbm, acc,
               didx_all, sidx0, sidx1, dst0, dst1, rows0, rows1,
               sem0, sem1, semi0, semi1):
    c = lax.axis_index("c")
    s = lax.axis_index("s")
    w = s * _NC + c
    tbase = w * (_CPT * _CH)

    def run_rel(x_hbm, si_hbm, di_hbm, r):
        def zero_rows(i, carry):
            for cc in range(_D // 16):
                rows0[i, pl.ds(cc * 16, 16)] = jnp.zeros((16,), jnp.float32)
            return carry
        lax.fori_loop(0, _CH, zero_rows, 0)
        for b in range(_STRIPE // _CH):
            pltpu.sync_copy(rows0, acc.at[pl.ds(s * _STRIPE + b * _CH, _CH)])

        pltpu.sync_copy(di_hbm.at[pl.ds(tbase, _CPT * _CH)], didx_all)
        plsc.subcore_barrier()

        def iload(j, sidx, semi):
            pltpu.async_copy(si_hbm.at[pl.ds(tbase + j * _CH, _CH)], sidx, semi)

        def idrain(sidx, semi):
            pltpu.make_async_copy(si_hbm.at[pl.ds(0, _CH)], sidx, semi).wait()

        def gather(sidx, rows, sem):
            pltpu.async_copy(x_hbm.at[sidx], rows, sem)

        def gdrain(rows, sem):
            pltpu.make_async_copy(x_hbm.at[pl.ds(0, _CH)], rows, sem).wait()

        def stage_dst(j, dst):
            def cp(i, carry):
                dst[pl.ds(i * 16, 16)] = didx_all[pl.ds(j * _CH + i * 16, 16)]
                return carry
            lax.fori_loop(0, _CH // 16, cp, 0)

        iload(0, sidx0, semi0)
        iload(1, sidx1, semi1)
        idrain(sidx0, semi0)
        gather(sidx0, rows0, sem0)
        stage_dst(0, dst0)
        idrain(sidx1, semi1)
        gather(sidx1, rows1, sem1)
        stage_dst(1, dst1)

        def body(j2, carry):
            j = j2 * 2
            gdrain(rows0, sem0)
            iload(j + 2, sidx0, semi0)
            pltpu.sync_copy(rows0, acc.at[dst0], add=True)
            idrain(sidx0, semi0)
            gather(sidx0, rows0, sem0)
            stage_dst(j + 2, dst0)
            gdrain(rows1, sem1)
            iload(j + 3, sidx1, semi1)
            pltpu.sync_copy(rows1, acc.at[dst1], add=True)
            idrain(sidx1, semi1)
            gather(sidx1, rows1, sem1)
            stage_dst(j + 3, dst1)
            return carry
        lax.fori_loop(0, _CPT // 2 - 1, body, 0)

        gdrain(rows0, sem0)
        pltpu.sync_copy(rows0, acc.at[dst0], add=True)
        gdrain(rows1, sem1)
        pltpu.sync_copy(rows1, acc.at[dst1], add=True)
        plsc.subcore_barrier()

        pltpu.sync_copy(
            acc.at[pl.ds(s * _STRIPE, _STRIPE)],
            out_hbm.at[pl.ds(r * (_NC * _NPAD) + c * _NPAD + s * _STRIPE,
                             _STRIPE)])

    run_rel(xn_hbm, si_nn, di_nn, 0)
    run_rel(xo_hbm, si_on, di_on, 1)
    run_rel(xo_hbm, si_oo, di_oo, 2)


def _seg3_call(xn, xo, si_nn, di_nn, si_on, di_on, si_oo, di_oo):
    out = pl.kernel(
        _seg3_body,
        out_type=jax.ShapeDtypeStruct((3 * _NC * _NPAD, _D), jnp.float32),
        mesh=_mesh(),
        scratch_types=[
            pltpu.VMEM_SHARED((_NPAD, _D), jnp.float32),
            pltpu.VMEM((_CPT * _CH,), jnp.int32),
            pltpu.VMEM((_CH,), jnp.int32),
            pltpu.VMEM((_CH,), jnp.int32),
            pltpu.VMEM((_CH,), jnp.int32),
            pltpu.VMEM((_CH,), jnp.int32),
            pltpu.VMEM((_CH, _D), jnp.float32),
            pltpu.VMEM((_CH, _D), jnp.float32),
            pltpu.SemaphoreType.DMA,
            pltpu.SemaphoreType.DMA,
            pltpu.SemaphoreType.DMA,
            pltpu.SemaphoreType.DMA,
        ],
    )(xn, xo, si_nn, di_nn, si_on, di_on, si_oo, di_oo)
    return out.reshape(3, _NC, _NPAD, _D)


def _stats_body(dnn, don, doo, dh, ef_hbm, out_hbm,
                acc, didx_all, dst, ef0, ef1, src, seme0, seme1):
    c = lax.axis_index("c")
    s = lax.axis_index("s")
    w = s * _NC + c
    tbase = w * (_CPT * _CH)
    ebase = w * (_CPT * _CH * _DE)
    zeros16 = jnp.zeros((16,), jnp.float32)
    ones16 = jnp.ones((16,), jnp.float32)

    def init_blk(i, carry):
        for cc in range(8):
            src[i, pl.ds(cc * 16, 16)] = zeros16
        return carry
    lax.fori_loop(0, _CH, init_blk, 0)
    for b in range(_STRIPE // _CH):
        pltpu.sync_copy(src, acc.at[pl.ds(s * _STRIPE + b * _CH, _CH)])
    plsc.subcore_barrier()

    def set_col(on_cc, off_ccs):
        def reset(i, carry):
            for cc in off_ccs:
                src[i, pl.ds(cc * 16, 16)] = zeros16
            src[i, pl.ds(on_cc * 16, 16)] = ones16
            return carry
        lax.fori_loop(0, _CH, reset, 0)

    def stage_dst(j):
        def cp(i, carry):
            dst[pl.ds(i * 16, 16)] = didx_all[pl.ds(j * _CH + i * 16, 16)]
            return carry
        lax.fori_loop(0, _CH // 16, cp, 0)

    def efload(j, efb, seme):
        pltpu.async_copy(
            ef_hbm.at[pl.ds(ebase + j * (_CH * _DE), _CH * _DE)], efb, seme)

    def efdrain(efb, seme):
        pltpu.make_async_copy(ef_hbm.at[pl.ds(0, _CH * _DE)], efb, seme).wait()

    def repack(efb):
        def rp(i, carry2):
            src[i, pl.ds(0, _DE)] = efb[pl.ds(i * _DE, _DE)]
            return carry2
        lax.fori_loop(0, _CH, rp, 0)

    set_col(1, ())
    pltpu.sync_copy(dh.at[pl.ds(tbase, _CPT * _CH)], didx_all)
    efload(0, ef0, seme0)
    efload(1, ef1, seme1)

    def body_h(j2, carry):
        j = j2 * 2
        efdrain(ef0, seme0)
        repack(ef0)
        efload(j + 2, ef0, seme0)
        stage_dst(j)
        pltpu.sync_copy(src, acc.at[dst], add=True)
        efdrain(ef1, seme1)
        repack(ef1)
        efload(j + 3, ef1, seme1)
        stage_dst(j + 1)
        pltpu.sync_copy(src, acc.at[dst], add=True)
        return carry
    lax.fori_loop(0, _CPT // 2 - 1, body_h, 0)
    for j in (_CPT - 2, _CPT - 1):
        efb, seme = (ef0, seme0) if j % 2 == 0 else (ef1, seme1)
        efdrain(efb, seme)
        repack(efb)
        stage_dst(j)
        pltpu.sync_copy(src, acc.at[dst], add=True)

    def count_phase(d_hbm, on_cc, off_ccs):
        set_col(on_cc, off_ccs)
        pltpu.sync_copy(d_hbm.at[pl.ds(tbase, _CPT * _CH)], didx_all)

        def body(j, carry):
            stage_dst(j)
            pltpu.sync_copy(src, acc.at[dst], add=True)
            return carry
        lax.fori_loop(0, _CPT, body, 0)

    count_phase(dnn, 2, (0, 1))
    count_phase(don, 4, (2,))
    count_phase(doo, 6, (4,))
    plsc.subcore_barrier()

    pltpu.sync_copy(acc.at[pl.ds(s * _STRIPE, _STRIPE)],
                    out_hbm.at[pl.ds(c * _NPAD + s * _STRIPE, _STRIPE)])


def _stats_call(dnn, don, doo, dh, ef_flat):
    out = pl.kernel(
        _stats_body,
        out_type=jax.ShapeDtypeStruct((_NC * _NPAD, _D), jnp.float32),
        mesh=_mesh(),
        scratch_types=[
            pltpu.VMEM_SHARED((_NPAD, _D), jnp.float32),
            pltpu.VMEM((_CPT * _CH,), jnp.int32),
            pltpu.VMEM((_CH,), jnp.int32),
            pltpu.VMEM((_CH * _DE,), jnp.float32),
            pltpu.VMEM((_CH * _DE,), jnp.float32),
            pltpu.VMEM((_CH, _D), jnp.float32),
            pltpu.SemaphoreType.DMA,
            pltpu.SemaphoreType.DMA,
        ],
    )(dnn, don, doo, dh, ef_flat)
    return out.reshape(_NC, _NPAD, _D)


_R = 1000


def _layer_body(relu,
                snn, son, soo, st,
                xn, xo,
                wlnn, wrnn, wlon, wron, wloo, wroo, weh, wnh,
                blnn, blon, bloo, beh, bnh,
                newo, oldo):
    f32 = jnp.float32
    stot = st[0] + st[1]

    def mean(sref, col):
        cnt = stot[:, col:col + 1]
        return (sref[0] + sref[1]) / jnp.clip(cnt, 1.0)

    mnn = mean(snn, 32)
    mon = mean(son, 64)
    moo = mean(soo, 96)
    new = (jnp.dot(mnn, wlnn[...], preferred_element_type=f32)
           + jnp.dot(mon, wlon[...], preferred_element_type=f32)
           + jnp.dot(xn[...], wrnn[...] + wron[...], preferred_element_type=f32)
           + blnn[...] + blon[...])
    ef_tot = stot[:, 0:_DE]
    cnt_h = stot[:, 16:17]
    old = (jnp.dot(moo, wloo[...], preferred_element_type=f32)
           + jnp.dot(xo[...], wroo[...] + wnh[...], preferred_element_type=f32)
           + jnp.dot(ef_tot, weh[...], preferred_element_type=f32)
           + cnt_h * beh[...] + bloo[...] + bnh[...])
    if relu:
        new = jnp.maximum(new, 0.0)
        old = jnp.maximum(old, 0.0)
    newo[...] = new
    oldo[...] = old


def _layer_call(relu, snn, son, soo, st, xn, xo,
                wlnn, wrnn, wlon, wron, wloo, wroo, weh, wnh,
                blnn, blon, bloo, beh, bnh):
    grid = (_N // _R,)

    def part3(width):
        return pl.BlockSpec((_NC, _R, width), lambda i: (0, i, 0))

    def rows(width):
        return pl.BlockSpec((_R, width), lambda i: (i, 0))

    def full(a, b):
        return pl.BlockSpec((a, b), lambda i: (0, 0))

    in_specs = [
        part3(_D), part3(_D), part3(_D), part3(_D),
        rows(_D), rows(_D),
        full(_D, _D), full(_D, _D), full(_D, _D), full(_D, _D),
        full(_D, _D), full(_D, _D), full(_DE, _D), full(_D, _D),
        full(1, _D), full(1, _D), full(1, _D), full(1, _D), full(1, _D),
    ]
    out_specs = [rows(_D), rows(_D)]
    out_shape = [jax.ShapeDtypeStruct((_N, _D), jnp.float32)] * 2
    return pl.pallas_call(
        functools.partial(_layer_body, relu),
        grid=grid,
        in_specs=in_specs,
        out_specs=out_specs,
        out_shape=out_shape,
    )(snn, son, soo, st, xn, xo,
      wlnn, wrnn, wlon, wron, wloo, wroo, weh, wnh,
      blnn, blon, bloo, beh, bnh)


def _pad_idx_tm(v, fill):
    v = v.astype(jnp.int32)
    pad = jnp.full((_EPAD - _E,), fill, jnp.int32)
    arr = jnp.concatenate([v, pad]).reshape(_CPT, _NTILES, _CH)
    return arr.transpose(1, 0, 2).reshape(-1)


def kernel(x_new, x_old, edge_feat_h, params,
           edge_index_nn, edge_index_on, edge_index_oo, edge_index_h):
    f32 = jnp.float32
    xn = x_new.astype(f32)
    xo = x_old.astype(f32)

    si_nn, di_nn = _pad_idx_tm(edge_index_nn[0], 0), _pad_idx_tm(edge_index_nn[1], _N)
    si_on, di_on = _pad_idx_tm(edge_index_on[0], 0), _pad_idx_tm(edge_index_on[1], _N)
    si_oo, di_oo = _pad_idx_tm(edge_index_oo[0], 0), _pad_idx_tm(edge_index_oo[1], _N)
    di_h = _pad_idx_tm(edge_index_h[1], _N)
    ef_pad = jnp.concatenate(
        [edge_feat_h.astype(f32),
         jnp.zeros((_EPAD - _E, _DE), f32)])
    ef_flat = (ef_pad.reshape(_CPT, _NTILES, _CH * _DE)
               .transpose(1, 0, 2).reshape(-1))

    st = _stats_call(di_nn, di_on, di_oo, di_h, ef_flat)

    b1 = lambda b: b.reshape(1, _D).astype(f32)
    for l in range(3):
        seg = _seg3_call(xn, xo, si_nn, di_nn, si_on, di_on, si_oo, di_oo)
        snn, son, soo = seg[0], seg[1], seg[2]
        xn, xo = _layer_call(
            l < 2, snn, son, soo, st, xn, xo,
            params['Wl_nn_%d' % l], params['Wr_nn_%d' % l],
            params['Wl_on_%d' % l], params['Wr_on_%d' % l],
            params['Wl_oo_%d' % l], params['Wr_oo_%d' % l],
            params['We_h_%d' % l], params['Wn_h_%d' % l],
            b1(params['bl_nn_%d' % l]), b1(params['bl_on_%d' % l]),
            b1(params['bl_oo_%d' % l]), b1(params['be_h_%d' % l]),
            b1(params['bn_h_%d' % l]))
    return xn, xo


# speedup vs baseline: 1.7697x; 1.7697x over previous
import functools

import jax
import jax.numpy as jnp
from jax import lax
from jax.experimental import pallas as pl
from jax.experimental.pallas import tpu as pltpu
from jax.experimental.pallas import tpu_sc as plsc

_N = 10000
_D = 128
_E = 320000
_DE = 16
_CH = 144
_NCHUNK = 2240
_EPAD = _NCHUNK * _CH
_NC, _NS = 2, 16
_NTILES = _NC * _NS
_CPT = _NCHUNK // _NTILES
_NPAD = 10240
_STRIPE = _NPAD // _NS


def _mesh():
    return plsc.VectorSubcoreMesh(
        core_axis_name="c", subcore_axis_name="s",
        num_cores=_NC, num_subcores=_NS)


def _seg_body(x_hbm, si_hbm, di_hbm, out_hbm, acc,
              didx_all, sidx0, sidx1, dst0, dst1, rows0, rows1,
              sem0, sem1, semi0, semi1):
    c = lax.axis_index("c")
    s = lax.axis_index("s")
    w = s * _NC + c
    tbase = w * (_CPT * _CH)

    def zero_rows(i, carry):
        for cc in range(_D // 16):
            rows0[i, pl.ds(cc * 16, 16)] = jnp.zeros((16,), jnp.float32)
        return carry
    lax.fori_loop(0, _CH, zero_rows, 0)
    for b in range(_STRIPE // 128):
        pltpu.sync_copy(rows0.at[pl.ds(0, 128)],
                        acc.at[pl.ds(s * _STRIPE + b * 128, 128)])

    pltpu.sync_copy(di_hbm.at[pl.ds(tbase, _CPT * _CH)], didx_all)
    plsc.subcore_barrier()

    def iload(j, sidx, semi):
        pltpu.async_copy(si_hbm.at[pl.ds(tbase + j * _CH, _CH)], sidx, semi)

    def idrain(sidx, semi):
        pltpu.make_async_copy(si_hbm.at[pl.ds(0, _CH)], sidx, semi).wait()

    def gather(sidx, rows, sem):
        pltpu.async_copy(x_hbm.at[sidx], rows, sem)

    def gdrain(rows, sem):
        pltpu.make_async_copy(x_hbm.at[pl.ds(0, _CH)], rows, sem).wait()

    def stage_dst(j, dst):
        def cp(i, carry):
            dst[pl.ds(i * 16, 16)] = didx_all[pl.ds(j * _CH + i * 16, 16)]
            return carry
        lax.fori_loop(0, _CH // 16, cp, 0)

    iload(0, sidx0, semi0)
    iload(1, sidx1, semi1)
    idrain(sidx0, semi0)
    gather(sidx0, rows0, sem0)
    stage_dst(0, dst0)
    idrain(sidx1, semi1)
    gather(sidx1, rows1, sem1)
    stage_dst(1, dst1)

    def body(j2, carry):
        j = j2 * 2
        gdrain(rows0, sem0)
        iload(j + 2, sidx0, semi0)
        pltpu.sync_copy(rows0, acc.at[dst0], add=True)
        idrain(sidx0, semi0)
        gather(sidx0, rows0, sem0)
        stage_dst(j + 2, dst0)
        gdrain(rows1, sem1)
        iload(j + 3, sidx1, semi1)
        pltpu.sync_copy(rows1, acc.at[dst1], add=True)
        idrain(sidx1, semi1)
        gather(sidx1, rows1, sem1)
        stage_dst(j + 3, dst1)
        return carry
    lax.fori_loop(0, _CPT // 2 - 1, body, 0)

    gdrain(rows0, sem0)
    pltpu.sync_copy(rows0, acc.at[dst0], add=True)
    gdrain(rows1, sem1)
    pltpu.sync_copy(rows1, acc.at[dst1], add=True)
    plsc.subcore_barrier()

    pltpu.sync_copy(acc.at[pl.ds(s * _STRIPE, _STRIPE)],
                    out_hbm.at[pl.ds(c * _NPAD + s * _STRIPE, _STRIPE)])


def _seg_call(x, si, di):
    out = pl.kernel(
        _seg_body,
        out_type=jax.ShapeDtypeStruct((_NC * _NPAD, _D), jnp.float32),
        mesh=_mesh(),
        scratch_types=[
            pltpu.VMEM_SHARED((_NPAD, _D), jnp.float32),
            pltpu.VMEM((_CPT * _CH,), jnp.int32),
            pltpu.VMEM((_CH,), jnp.int32),
            pltpu.VMEM((_CH,), jnp.int32),
            pltpu.VMEM((_CH,), jnp.int32),
            pltpu.VMEM((_CH,), jnp.int32),
            pltpu.VMEM((_CH, _D), jnp.float32),
            pltpu.VMEM((_CH, _D), jnp.float32),
            pltpu.SemaphoreType.DMA,
            pltpu.SemaphoreType.DMA,
            pltpu.SemaphoreType.DMA,
            pltpu.SemaphoreType.DMA,
        ],
    )(x, si, di)
    return out.reshape(_NC, _NPAD, _D)


def _stats_body(dnn, don, doo, dh, ef_hbm, out_hbm,
                acc, didx_all, dst, ef0, ef1, src, seme0, seme1):
    c = lax.axis_index("c")
    s = lax.axis_index("s")
    w = s * _NC + c
    tbase = w * (_CPT * _CH)
    ebase = w * (_CPT * _CH * _DE)
    zeros16 = jnp.zeros((16,), jnp.float32)
    ones16 = jnp.ones((16,), jnp.float32)

    def init_blk(i, carry):
        for cc in range(8):
            src[i, pl.ds(cc * 16, 16)] = zeros16
        return carry
    lax.fori_loop(0, _CH, init_blk, 0)
    for b in range(_STRIPE // 128):
        pltpu.sync_copy(src.at[pl.ds(0, 128)],
                        acc.at[pl.ds(s * _STRIPE + b * 128, 128)])
    plsc.subcore_barrier()

    def set_col(on_cc, off_ccs):
        def reset(i, carry):
            for cc in off_ccs:
                src[i, pl.ds(cc * 16, 16)] = zeros16
            src[i, pl.ds(on_cc * 16, 16)] = ones16
            return carry
        lax.fori_loop(0, _CH, reset, 0)

    def stage_dst(j):
        def cp(i, carry):
            dst[pl.ds(i * 16, 16)] = didx_all[pl.ds(j * _CH + i * 16, 16)]
            return carry
        lax.fori_loop(0, _CH // 16, cp, 0)

    def efload(j, efb, seme):
        pltpu.async_copy(
            ef_hbm.at[pl.ds(ebase + j * (_CH * _DE), _CH * _DE)], efb, seme)

    def efdrain(efb, seme):
        pltpu.make_async_copy(ef_hbm.at[pl.ds(0, _CH * _DE)], efb, seme).wait()

    def repack(efb):
        def rp(i, carry2):
            src[i, pl.ds(0, _DE)] = efb[pl.ds(i * _DE, _DE)]
            return carry2
        lax.fori_loop(0, _CH, rp, 0)

    set_col(1, ())
    pltpu.sync_copy(dh.at[pl.ds(tbase, _CPT * _CH)], didx_all)
    efload(0, ef0, seme0)
    efload(1, ef1, seme1)

    def body_h(j2, carry):
        j = j2 * 2
        efdrain(ef0, seme0)
        repack(ef0)
        efload(j + 2, ef0, seme0)
        stage_dst(j)
        pltpu.sync_copy(src, acc.at[dst], add=True)
        efdrain(ef1, seme1)
        repack(ef1)
        efload(j + 3, ef1, seme1)
        stage_dst(j + 1)
        pltpu.sync_copy(src, acc.at[dst], add=True)
        return carry
    lax.fori_loop(0, _CPT // 2 - 1, body_h, 0)
    for j in (_CPT - 2, _CPT - 1):
        efb, seme = (ef0, seme0) if j % 2 == 0 else (ef1, seme1)
        efdrain(efb, seme)
        repack(efb)
        stage_dst(j)
        pltpu.sync_copy(src, acc.at[dst], add=True)

    def count_phase(d_hbm, on_cc, off_ccs):
        set_col(on_cc, off_ccs)
        pltpu.sync_copy(d_hbm.at[pl.ds(tbase, _CPT * _CH)], didx_all)

        def body(j, carry):
            stage_dst(j)
            pltpu.sync_copy(src, acc.at[dst], add=True)
            return carry
        lax.fori_loop(0, _CPT, body, 0)

    count_phase(dnn, 2, (0, 1))
    count_phase(don, 4, (2,))
    count_phase(doo, 6, (4,))
    plsc.subcore_barrier()

    pltpu.sync_copy(acc.at[pl.ds(s * _STRIPE, _STRIPE)],
                    out_hbm.at[pl.ds(c * _NPAD + s * _STRIPE, _STRIPE)])


def _stats_call(dnn, don, doo, dh, ef_flat):
    out = pl.kernel(
        _stats_body,
        out_type=jax.ShapeDtypeStruct((_NC * _NPAD, _D), jnp.float32),
        mesh=_mesh(),
        scratch_types=[
            pltpu.VMEM_SHARED((_NPAD, _D), jnp.float32),
            pltpu.VMEM((_CPT * _CH,), jnp.int32),
            pltpu.VMEM((_CH,), jnp.int32),
            pltpu.VMEM((_CH * _DE,), jnp.float32),
            pltpu.VMEM((_CH * _DE,), jnp.float32),
            pltpu.VMEM((_CH, _D), jnp.float32),
            pltpu.SemaphoreType.DMA,
            pltpu.SemaphoreType.DMA,
        ],
    )(dnn, don, doo, dh, ef_flat)
    return out.reshape(_NC, _NPAD, _D)


_R = 1000


def _layer_body(relu,
                snn, son, soo, st,
                xn, xo,
                wlnn, wrnn, wlon, wron, wloo, wroo, weh, wnh,
                blnn, blon, bloo, beh, bnh,
                newo, oldo):
    f32 = jnp.float32
    stot = st[0] + st[1]

    def mean(sref, col):
        cnt = stot[:, col:col + 1]
        return (sref[0] + sref[1]) / jnp.clip(cnt, 1.0)

    mnn = mean(snn, 32)
    mon = mean(son, 64)
    moo = mean(soo, 96)
    new = (jnp.dot(mnn, wlnn[...], preferred_element_type=f32)
           + jnp.dot(mon, wlon[...], preferred_element_type=f32)
           + jnp.dot(xn[...], wrnn[...] + wron[...], preferred_element_type=f32)
           + blnn[...] + blon[...])
    ef_tot = stot[:, 0:_DE]
    cnt_h = stot[:, 16:17]
    old = (jnp.dot(moo, wloo[...], preferred_element_type=f32)
           + jnp.dot(xo[...], wroo[...] + wnh[...], preferred_element_type=f32)
           + jnp.dot(ef_tot, weh[...], preferred_element_type=f32)
           + cnt_h * beh[...] + bloo[...] + bnh[...])
    if relu:
        new = jnp.maximum(new, 0.0)
        old = jnp.maximum(old, 0.0)
    newo[...] = new
    oldo[...] = old


def _layer_call(relu, snn, son, soo, st, xn, xo,
                wlnn, wrnn, wlon, wron, wloo, wroo, weh, wnh,
                blnn, blon, bloo, beh, bnh):
    grid = (_N // _R,)

    def part3(width):
        return pl.BlockSpec((_NC, _R, width), lambda i: (0, i, 0))

    def rows(width):
        return pl.BlockSpec((_R, width), lambda i: (i, 0))

    def full(a, b):
        return pl.BlockSpec((a, b), lambda i: (0, 0))

    in_specs = [
        part3(_D), part3(_D), part3(_D), part3(_D),
        rows(_D), rows(_D),
        full(_D, _D), full(_D, _D), full(_D, _D), full(_D, _D),
        full(_D, _D), full(_D, _D), full(_DE, _D), full(_D, _D),
        full(1, _D), full(1, _D), full(1, _D), full(1, _D), full(1, _D),
    ]
    out_specs = [rows(_D), rows(_D)]
    out_shape = [jax.ShapeDtypeStruct((_N, _D), jnp.float32)] * 2
    return pl.pallas_call(
        functools.partial(_layer_body, relu),
        grid=grid,
        in_specs=in_specs,
        out_specs=out_specs,
        out_shape=out_shape,
    )(snn, son, soo, st, xn, xo,
      wlnn, wrnn, wlon, wron, wloo, wroo, weh, wnh,
      blnn, blon, bloo, beh, bnh)


def _pad_idx_tm(v, fill):
    v = v.astype(jnp.int32)
    pad = jnp.full((_EPAD - _E,), fill, jnp.int32)
    arr = jnp.concatenate([v, pad]).reshape(_CPT, _NTILES, _CH)
    return arr.transpose(1, 0, 2).reshape(-1)


def kernel(x_new, x_old, edge_feat_h, params,
           edge_index_nn, edge_index_on, edge_index_oo, edge_index_h):
    f32 = jnp.float32
    xn = x_new.astype(f32)
    xo = x_old.astype(f32)

    si_nn, di_nn = _pad_idx_tm(edge_index_nn[0], 0), _pad_idx_tm(edge_index_nn[1], _N)
    si_on, di_on = _pad_idx_tm(edge_index_on[0], 0), _pad_idx_tm(edge_index_on[1], _N)
    si_oo, di_oo = _pad_idx_tm(edge_index_oo[0], 0), _pad_idx_tm(edge_index_oo[1], _N)
    di_h = _pad_idx_tm(edge_index_h[1], _N)
    ef_pad = jnp.concatenate(
        [edge_feat_h.astype(f32),
         jnp.zeros((_EPAD - _E, _DE), f32)])
    ef_flat = (ef_pad.reshape(_CPT, _NTILES, _CH * _DE)
               .transpose(1, 0, 2).reshape(-1))

    st = _stats_call(di_nn, di_on, di_oo, di_h, ef_flat)

    b1 = lambda b: b.reshape(1, _D).astype(f32)
    for l in range(3):
        snn = _seg_call(xn, si_nn, di_nn)
        son = _seg_call(xo, si_on, di_on)
        soo = _seg_call(xo, si_oo, di_oo)
        xn, xo = _layer_call(
            l < 2, snn, son, soo, st, xn, xo,
            params['Wl_nn_%d' % l], params['Wr_nn_%d' % l],
            params['Wl_on_%d' % l], params['Wr_on_%d' % l],
            params['Wl_oo_%d' % l], params['Wr_oo_%d' % l],
            params['We_h_%d' % l], params['Wn_h_%d' % l],
            b1(params['bl_nn_%d' % l]), b1(params['bl_on_%d' % l]),
            b1(params['bl_oo_%d' % l]), b1(params['be_h_%d' % l]),
            b1(params['bn_h_%d' % l]))
    return xn, xo


# speedup vs baseline: 1.7717x; 1.0011x over previous
import functools

import jax
import jax.numpy as jnp
from jax import lax
from jax.experimental import pallas as pl
from jax.experimental.pallas import tpu as pltpu
from jax.experimental.pallas import tpu_sc as plsc

_N = 10000
_D = 128
_E = 320000
_DE = 16
_CH = 144
_NCHUNK = 2240
_EPAD = _NCHUNK * _CH
_NC, _NS = 2, 16
_NTILES = _NC * _NS
_CPT = _NCHUNK // _NTILES
_NPAD = 10240
_STRIPE = _NPAD // _NS


def _mesh():
    return plsc.VectorSubcoreMesh(
        core_axis_name="c", subcore_axis_name="s",
        num_cores=_NC, num_subcores=_NS)


def _seg_body(x_hbm, si_hbm, di_hbm, out_hbm, acc,
              sidx0, sidx1, dst0, dst1, rows0, rows1,
              sem0, sem1, semi0, semi1, semd0, semd1):
    c = lax.axis_index("c")
    s = lax.axis_index("s")
    w = s * _NC + c
    tbase = w * (_CPT * _CH)

    def zero_rows(i, carry):
        for cc in range(_D // 16):
            rows0[i, pl.ds(cc * 16, 16)] = jnp.zeros((16,), jnp.float32)
        return carry
    lax.fori_loop(0, _CH, zero_rows, 0)
    for b in range(_STRIPE // 128):
        pltpu.sync_copy(rows0.at[pl.ds(0, 128)],
                        acc.at[pl.ds(s * _STRIPE + b * 128, 128)])
    plsc.subcore_barrier()

    def iload(j, sidx, semi):
        pltpu.async_copy(si_hbm.at[pl.ds(tbase + j * _CH, _CH)], sidx, semi)

    def idrain(sidx, semi):
        pltpu.make_async_copy(si_hbm.at[pl.ds(0, _CH)], sidx, semi).wait()

    def dload(j, dst, semd):
        pltpu.async_copy(di_hbm.at[pl.ds(tbase + j * _CH, _CH)], dst, semd)

    def ddrain(dst, semd):
        pltpu.make_async_copy(di_hbm.at[pl.ds(0, _CH)], dst, semd).wait()

    def gather(sidx, rows, sem):
        pltpu.async_copy(x_hbm.at[sidx], rows, sem)

    def gdrain(rows, sem):
        pltpu.make_async_copy(x_hbm.at[pl.ds(0, _CH)], rows, sem).wait()

    iload(0, sidx0, semi0)
    dload(0, dst0, semd0)
    iload(1, sidx1, semi1)
    dload(1, dst1, semd1)
    idrain(sidx0, semi0)
    gather(sidx0, rows0, sem0)
    idrain(sidx1, semi1)
    gather(sidx1, rows1, sem1)
    ddrain(dst0, semd0)
    ddrain(dst1, semd1)

    def body(j2, carry):
        j = j2 * 2
        gdrain(rows0, sem0)
        iload(j + 2, sidx0, semi0)
        pltpu.sync_copy(rows0, acc.at[dst0], add=True)
        dload(j + 2, dst0, semd0)
        idrain(sidx0, semi0)
        gather(sidx0, rows0, sem0)
        gdrain(rows1, sem1)
        iload(j + 3, sidx1, semi1)
        pltpu.sync_copy(rows1, acc.at[dst1], add=True)
        dload(j + 3, dst1, semd1)
        idrain(sidx1, semi1)
        gather(sidx1, rows1, sem1)
        ddrain(dst0, semd0)
        ddrain(dst1, semd1)
        return carry
    lax.fori_loop(0, _CPT // 2 - 1, body, 0)

    gdrain(rows0, sem0)
    pltpu.sync_copy(rows0, acc.at[dst0], add=True)
    gdrain(rows1, sem1)
    pltpu.sync_copy(rows1, acc.at[dst1], add=True)
    plsc.subcore_barrier()

    pltpu.sync_copy(acc.at[pl.ds(s * _STRIPE, _STRIPE)],
                    out_hbm.at[pl.ds(c * _NPAD + s * _STRIPE, _STRIPE)])


def _seg_call(x, si, di):
    out = pl.kernel(
        _seg_body,
        out_type=jax.ShapeDtypeStruct((_NC * _NPAD, _D), jnp.float32),
        mesh=_mesh(),
        scratch_types=[
            pltpu.VMEM_SHARED((_NPAD, _D), jnp.float32),
            pltpu.VMEM((_CH,), jnp.int32),
            pltpu.VMEM((_CH,), jnp.int32),
            pltpu.VMEM((_CH,), jnp.int32),
            pltpu.VMEM((_CH,), jnp.int32),
            pltpu.VMEM((_CH, _D), jnp.float32),
            pltpu.VMEM((_CH, _D), jnp.float32),
            pltpu.SemaphoreType.DMA,
            pltpu.SemaphoreType.DMA,
            pltpu.SemaphoreType.DMA,
            pltpu.SemaphoreType.DMA,
            pltpu.SemaphoreType.DMA,
            pltpu.SemaphoreType.DMA,
        ],
    )(x, si, di)
    return out.reshape(_NC, _NPAD, _D)


def _stats_body(dnn, don, doo, dh, ef_hbm, out_hbm,
                acc, didx_all, dst, ef0, ef1, src, seme0, seme1):
    c = lax.axis_index("c")
    s = lax.axis_index("s")
    w = s * _NC + c
    tbase = w * (_CPT * _CH)
    ebase = w * (_CPT * _CH * _DE)
    zeros16 = jnp.zeros((16,), jnp.float32)
    ones16 = jnp.ones((16,), jnp.float32)

    def init_blk(i, carry):
        for cc in range(8):
            src[i, pl.ds(cc * 16, 16)] = zeros16
        return carry
    lax.fori_loop(0, _CH, init_blk, 0)
    for b in range(_STRIPE // 128):
        pltpu.sync_copy(src.at[pl.ds(0, 128)],
                        acc.at[pl.ds(s * _STRIPE + b * 128, 128)])
    plsc.subcore_barrier()

    def set_col(on_cc, off_ccs):
        def reset(i, carry):
            for cc in off_ccs:
                src[i, pl.ds(cc * 16, 16)] = zeros16
            src[i, pl.ds(on_cc * 16, 16)] = ones16
            return carry
        lax.fori_loop(0, _CH, reset, 0)

    def stage_dst(j):
        def cp(i, carry):
            dst[pl.ds(i * 16, 16)] = didx_all[pl.ds(j * _CH + i * 16, 16)]
            return carry
        lax.fori_loop(0, _CH // 16, cp, 0)

    def efload(j, efb, seme):
        pltpu.async_copy(
            ef_hbm.at[pl.ds(ebase + j * (_CH * _DE), _CH * _DE)], efb, seme)

    def efdrain(efb, seme):
        pltpu.make_async_copy(ef_hbm.at[pl.ds(0, _CH * _DE)], efb, seme).wait()

    def repack(efb):
        def rp(i, carry2):
            src[i, pl.ds(0, _DE)] = efb[pl.ds(i * _DE, _DE)]
            return carry2
        lax.fori_loop(0, _CH, rp, 0)

    set_col(1, ())
    pltpu.sync_copy(dh.at[pl.ds(tbase, _CPT * _CH)], didx_all)
    efload(0, ef0, seme0)
    efload(1, ef1, seme1)

    def body_h(j2, carry):
        j = j2 * 2
        efdrain(ef0, seme0)
        repack(ef0)
        efload(j + 2, ef0, seme0)
        stage_dst(j)
        pltpu.sync_copy(src, acc.at[dst], add=True)
        efdrain(ef1, seme1)
        repack(ef1)
        efload(j + 3, ef1, seme1)
        stage_dst(j + 1)
        pltpu.sync_copy(src, acc.at[dst], add=True)
        return carry
    lax.fori_loop(0, _CPT // 2 - 1, body_h, 0)
    for j in (_CPT - 2, _CPT - 1):
        efb, seme = (ef0, seme0) if j % 2 == 0 else (ef1, seme1)
        efdrain(efb, seme)
        repack(efb)
        stage_dst(j)
        pltpu.sync_copy(src, acc.at[dst], add=True)

    def count_phase(d_hbm, on_cc, off_ccs):
        set_col(on_cc, off_ccs)
        pltpu.sync_copy(d_hbm.at[pl.ds(tbase, _CPT * _CH)], didx_all)

        def body(j, carry):
            stage_dst(j)
            pltpu.sync_copy(src, acc.at[dst], add=True)
            return carry
        lax.fori_loop(0, _CPT, body, 0)

    count_phase(dnn, 2, (0, 1))
    count_phase(don, 4, (2,))
    count_phase(doo, 6, (4,))
    plsc.subcore_barrier()

    pltpu.sync_copy(acc.at[pl.ds(s * _STRIPE, _STRIPE)],
                    out_hbm.at[pl.ds(c * _NPAD + s * _STRIPE, _STRIPE)])


def _stats_call(dnn, don, doo, dh, ef_flat):
    out = pl.kernel(
        _stats_body,
        out_type=jax.ShapeDtypeStruct((_NC * _NPAD, _D), jnp.float32),
        mesh=_mesh(),
        scratch_types=[
            pltpu.VMEM_SHARED((_NPAD, _D), jnp.float32),
            pltpu.VMEM((_CPT * _CH,), jnp.int32),
            pltpu.VMEM((_CH,), jnp.int32),
            pltpu.VMEM((_CH * _DE,), jnp.float32),
            pltpu.VMEM((_CH * _DE,), jnp.float32),
            pltpu.VMEM((_CH, _D), jnp.float32),
            pltpu.SemaphoreType.DMA,
            pltpu.SemaphoreType.DMA,
        ],
    )(dnn, don, doo, dh, ef_flat)
    return out.reshape(_NC, _NPAD, _D)


_R = 1000


def _layer_body(relu,
                snn, son, soo, st,
                xn, xo,
                wlnn, wrnn, wlon, wron, wloo, wroo, weh, wnh,
                blnn, blon, bloo, beh, bnh,
                newo, oldo):
    f32 = jnp.float32
    stot = st[0] + st[1]

    def mean(sref, col):
        cnt = stot[:, col:col + 1]
        return (sref[0] + sref[1]) / jnp.clip(cnt, 1.0)

    mnn = mean(snn, 32)
    mon = mean(son, 64)
    moo = mean(soo, 96)
    new = (jnp.dot(mnn, wlnn[...], preferred_element_type=f32)
           + jnp.dot(mon, wlon[...], preferred_element_type=f32)
           + jnp.dot(xn[...], wrnn[...] + wron[...], preferred_element_type=f32)
           + blnn[...] + blon[...])
    ef_tot = stot[:, 0:_DE]
    cnt_h = stot[:, 16:17]
    old = (jnp.dot(moo, wloo[...], preferred_element_type=f32)
           + jnp.dot(xo[...], wroo[...] + wnh[...], preferred_element_type=f32)
           + jnp.dot(ef_tot, weh[...], preferred_element_type=f32)
           + cnt_h * beh[...] + bloo[...] + bnh[...])
    if relu:
        new = jnp.maximum(new, 0.0)
        old = jnp.maximum(old, 0.0)
    newo[...] = new
    oldo[...] = old


def _layer_call(relu, snn, son, soo, st, xn, xo,
                wlnn, wrnn, wlon, wron, wloo, wroo, weh, wnh,
                blnn, blon, bloo, beh, bnh):
    grid = (_N // _R,)

    def part3(width):
        return pl.BlockSpec((_NC, _R, width), lambda i: (0, i, 0))

    def rows(width):
        return pl.BlockSpec((_R, width), lambda i: (i, 0))

    def full(a, b):
        return pl.BlockSpec((a, b), lambda i: (0, 0))

    in_specs = [
        part3(_D), part3(_D), part3(_D), part3(_D),
        rows(_D), rows(_D),
        full(_D, _D), full(_D, _D), full(_D, _D), full(_D, _D),
        full(_D, _D), full(_D, _D), full(_DE, _D), full(_D, _D),
        full(1, _D), full(1, _D), full(1, _D), full(1, _D), full(1, _D),
    ]
    out_specs = [rows(_D), rows(_D)]
    out_shape = [jax.ShapeDtypeStruct((_N, _D), jnp.float32)] * 2
    return pl.pallas_call(
        functools.partial(_layer_body, relu),
        grid=grid,
        in_specs=in_specs,
        out_specs=out_specs,
        out_shape=out_shape,
    )(snn, son, soo, st, xn, xo,
      wlnn, wrnn, wlon, wron, wloo, wroo, weh, wnh,
      blnn, blon, bloo, beh, bnh)


def _pad_idx_tm(v, fill):
    v = v.astype(jnp.int32)
    pad = jnp.full((_EPAD - _E,), fill, jnp.int32)
    arr = jnp.concatenate([v, pad]).reshape(_CPT, _NTILES, _CH)
    return arr.transpose(1, 0, 2).reshape(-1)


def kernel(x_new, x_old, edge_feat_h, params,
           edge_index_nn, edge_index_on, edge_index_oo, edge_index_h):
    f32 = jnp.float32
    xn = x_new.astype(f32)
    xo = x_old.astype(f32)

    si_nn, di_nn = _pad_idx_tm(edge_index_nn[0], 0), _pad_idx_tm(edge_index_nn[1], _N)
    si_on, di_on = _pad_idx_tm(edge_index_on[0], 0), _pad_idx_tm(edge_index_on[1], _N)
    si_oo, di_oo = _pad_idx_tm(edge_index_oo[0], 0), _pad_idx_tm(edge_index_oo[1], _N)
    di_h = _pad_idx_tm(edge_index_h[1], _N)
    ef_pad = jnp.concatenate(
        [edge_feat_h.astype(f32),
         jnp.zeros((_EPAD - _E, _DE), f32)])
    ef_flat = (ef_pad.reshape(_CPT, _NTILES, _CH * _DE)
               .transpose(1, 0, 2).reshape(-1))

    st = _stats_call(di_nn, di_on, di_oo, di_h, ef_flat)

    b1 = lambda b: b.reshape(1, _D).astype(f32)
    for l in range(3):
        snn = _seg_call(xn, si_nn, di_nn)
        son = _seg_call(xo, si_on, di_on)
        soo = _seg_call(xo, si_oo, di_oo)
        xn, xo = _layer_call(
            l < 2, snn, son, soo, st, xn, xo,
            params['Wl_nn_%d' % l], params['Wr_nn_%d' % l],
            params['Wl_on_%d' % l], params['Wr_on_%d' % l],
            params['Wl_oo_%d' % l], params['Wr_oo_%d' % l],
            params['We_h_%d' % l], params['Wn_h_%d' % l],
            b1(params['bl_nn_%d' % l]), b1(params['bl_on_%d' % l]),
            b1(params['bl_oo_%d' % l]), b1(params['be_h_%d' % l]),
            b1(params['bn_h_%d' % l]))
    return xn, xo
